# baseline (speedup 1.0000x reference)
import functools

import jax
import jax.numpy as jnp
from jax import lax
from jax.experimental import pallas as pl
from jax.experimental.pallas import tpu as pltpu
from jax.experimental.pallas import tpu_sc as plsc

IN_DIM = 1024
EMBED = 64
NTOK = 32
KCODES = 1024
BATCH = 4096
HID = 512

BLK = 512
GRID = BATCH // BLK

NC = 2
NS = 16
NW = NC * NS
NROWS = BATCH * NTOK
B_PER_W = NROWS // NW
CHUNK = 512
N_CHUNK = B_PER_W // CHUNK
EMB_PAD = 128

_INV_SQRT2 = 0.7071067811865476


def _gelu(v):
    return 0.5 * v * (1.0 + jax.lax.erf(v * _INV_SQRT2))


def _dot(a, b):
    return jnp.dot(a, b, preferred_element_type=jnp.float32)


def _enc_kernel(x_ref, eW1, eb1, eW2, eb2, eW3, eb3, cbT_ref,
                idx_ref, closs_ref):
    x = x_ref[...]
    h = _gelu(_dot(x, eW1[...]) + eb1[...])
    h = _gelu(_dot(h, eW2[...]) + eb2[...])
    z = _dot(h, eW3[...]) + eb3[...]

    cbT = cbT_ref[...]
    cb_sq = jnp.sum(cbT * cbT, axis=0, keepdims=True)
    cbT2 = cbT * -2.0

    tt = jax.lax.broadcasted_iota(jnp.int32, (BLK, NTOK), 1)

    idx_mat = jnp.zeros((BLK, NTOK), dtype=jnp.int32)
    closs_acc = jnp.float32(0.0)
    for t in range(NTOK):
        f = z[:, EMBED * t:EMBED * (t + 1)]
        f_sq = jnp.sum(f * f, axis=1, keepdims=True)
        d = (f_sq + _dot(f, cbT2)) + cb_sq
        idx_t = jnp.argmin(d, axis=1).astype(jnp.int32)
        idx_mat = jnp.where(tt == t, idx_t[:, None], idx_mat)
        closs_acc += jnp.sum(jnp.min(d, axis=1))

    idx_ref[...] = idx_mat
    closs_ref[...] = jnp.broadcast_to(closs_acc, (1, 1, 128))


def _dec_kernel(qz_ref, dW1, db1, dW2, db2, dW3, db3, rec_ref):
    qz = qz_ref[...]
    r = _gelu(_dot(qz, dW1[...]) + db1[...])
    r = _gelu(_dot(r, dW2[...]) + db2[...])
    rec_ref[...] = _dot(r, dW3[...]) + db3[...]


def _sc_gather(cb_hbm, idx_hbm, out_hbm, idx_v, rows_v, sem):
    wid = lax.axis_index("s") * NC + lax.axis_index("c")
    base = wid * B_PER_W
    for c in range(N_CHUNK):
        off = base + c * CHUNK
        pltpu.sync_copy(idx_hbm.at[pl.ds(off, CHUNK)], idx_v)
        pltpu.async_copy(cb_hbm.at[idx_v], rows_v, sem).wait()
        pltpu.sync_copy(rows_v, out_hbm.at[pl.ds(off, CHUNK)])


def kernel(x, enc_W1, enc_b1, enc_W2, enc_b2, enc_W3, enc_b3,
           dec_W1, dec_b1, dec_W2, dec_b2, dec_W3, dec_b3, codebook):
    cbT = codebook.T
    full = lambda shape: pl.BlockSpec(shape, lambda i: (0, 0))
    row = lambda n: pl.BlockSpec((1, n), lambda i: (0, 0))

    idx, closs = pl.pallas_call(
        _enc_kernel,
        grid=(GRID,),
        in_specs=[
            pl.BlockSpec((BLK, IN_DIM), lambda i: (i, 0)),
            full((IN_DIM, HID)), row(HID),
            full((HID, HID)), row(HID),
            full((HID, EMBED * NTOK)), row(EMBED * NTOK),
            full((EMBED, KCODES)),
        ],
        out_specs=[
            pl.BlockSpec((BLK, NTOK), lambda i: (i, 0)),
            pl.BlockSpec((1, 1, 128), lambda i: (i, 0, 0)),
        ],
        out_shape=[
            jax.ShapeDtypeStruct((BATCH, NTOK), jnp.int32),
            jax.ShapeDtypeStruct((GRID, 1, 128), jnp.float32),
        ],
        compiler_params=pltpu.CompilerParams(
            dimension_semantics=("parallel",),
        ),
    )(x,
      enc_W1, enc_b1.reshape(1, HID),
      enc_W2, enc_b2.reshape(1, HID),
      enc_W3, enc_b3.reshape(1, EMBED * NTOK),
      cbT)

    gather = functools.partial(
        pl.kernel,
        mesh=plsc.VectorSubcoreMesh(core_axis_name="c", subcore_axis_name="s"),
        out_type=jax.ShapeDtypeStruct((NROWS, EMB_PAD), jnp.float32),
        scratch_types=[
            pltpu.VMEM((CHUNK,), jnp.int32),
            pltpu.VMEM((CHUNK, EMB_PAD), jnp.float32),
            pltpu.SemaphoreType.DMA,
        ],
    )(_sc_gather)
    cb_pad = jnp.pad(codebook, ((0, 0), (0, EMB_PAD - EMBED)))
    q_rows = gather(cb_pad, idx.reshape(NROWS))
    qz = q_rows.reshape(BATCH, NTOK * EMB_PAD)

    dW1_pad = jnp.pad(dec_W1.reshape(NTOK, EMBED, HID),
                      ((0, 0), (0, EMB_PAD - EMBED), (0, 0))
                      ).reshape(NTOK * EMB_PAD, HID)

    rec = pl.pallas_call(
        _dec_kernel,
        grid=(GRID,),
        in_specs=[
            pl.BlockSpec((BLK, EMB_PAD * NTOK), lambda i: (i, 0)),
            full((EMB_PAD * NTOK, HID)), row(HID),
            full((HID, HID)), row(HID),
            full((HID, IN_DIM)), row(IN_DIM),
        ],
        out_specs=pl.BlockSpec((BLK, IN_DIM), lambda i: (i, 0)),
        out_shape=jax.ShapeDtypeStruct((BATCH, IN_DIM), jnp.float32),
        compiler_params=pltpu.CompilerParams(
            dimension_semantics=("parallel",),
        ),
    )(qz,
      dW1_pad, dec_b1.reshape(1, HID),
      dec_W2, dec_b2.reshape(1, HID),
      dec_W3, dec_b3.reshape(1, IN_DIM))

    commit_loss = jnp.sum(closs[:, 0, 0]) / jnp.float32(BATCH * NTOK * EMBED)
    return rec, idx, commit_loss


# speedup vs baseline: 20.4001x; 20.4001x over previous
import jax
import jax.numpy as jnp
from jax.experimental import pallas as pl
from jax.experimental.pallas import tpu as pltpu

IN_DIM = 1024
EMBED = 64
NTOK = 32
KCODES = 1024
BATCH = 4096
HID = 512

BLK = 512
GRID = BATCH // BLK

_INV_SQRT2 = 0.7071067811865476


def _gelu(v):
    return 0.5 * v * (1.0 + jax.lax.erf(v * _INV_SQRT2))


def _dot(a, b):
    return jnp.dot(a, b, preferred_element_type=jnp.float32)


def _fused_kernel(x_ref, eW1, eb1, eW2, eb2, eW3, eb3,
                  dW1, db1, dW2, db2, dW3, db3,
                  cb_ref, cbT_ref,
                  rec_ref, idx_ref, closs_ref,
                  qz_ref):
    x = x_ref[...]
    h = _gelu(_dot(x, eW1[...]) + eb1[...])
    h = _gelu(_dot(h, eW2[...]) + eb2[...])
    z = _dot(h, eW3[...]) + eb3[...]

    cb = cb_ref[...]
    cbT = cbT_ref[...]
    cb_sq = jnp.sum(cbT * cbT, axis=0, keepdims=True)
    cbT2 = cbT * -2.0

    ii = jax.lax.broadcasted_iota(jnp.int32, (BLK, KCODES), 1)
    tt = jax.lax.broadcasted_iota(jnp.int32, (BLK, NTOK), 1)

    idx_mat = jnp.zeros((BLK, NTOK), dtype=jnp.int32)
    closs_acc = jnp.float32(0.0)
    for t in range(NTOK):
        f = z[:, EMBED * t:EMBED * (t + 1)]
        f_sq = jnp.sum(f * f, axis=1, keepdims=True)
        d = (f_sq + _dot(f, cbT2)) + cb_sq
        idx_t = jnp.argmin(d, axis=1).astype(jnp.int32)
        idx_mat = jnp.where(tt == t, idx_t[:, None], idx_mat)
        onehot = (ii == idx_t[:, None]).astype(jnp.float32)
        q = _dot(onehot, cb)
        closs_acc += jnp.sum((q - f) ** 2)
        qz_ref[:, EMBED * t:EMBED * (t + 1)] = q

    idx_ref[...] = idx_mat

    qz = qz_ref[...]
    r = _gelu(_dot(qz, dW1[...]) + db1[...])
    r = _gelu(_dot(r, dW2[...]) + db2[...])
    rec_ref[...] = _dot(r, dW3[...]) + db3[...]

    closs_ref[...] = jnp.broadcast_to(closs_acc, (1, 1, 128))


def kernel(x, enc_W1, enc_b1, enc_W2, enc_b2, enc_W3, enc_b3,
           dec_W1, dec_b1, dec_W2, dec_b2, dec_W3, dec_b3, codebook):
    cbT = codebook.T
    full = lambda shape: pl.BlockSpec(shape, lambda i: (0, 0))
    row = lambda n: pl.BlockSpec((1, n), lambda i: (0, 0))

    rec, idx, closs = pl.pallas_call(
        _fused_kernel,
        grid=(GRID,),
        in_specs=[
            pl.BlockSpec((BLK, IN_DIM), lambda i: (i, 0)),
            full((IN_DIM, HID)), row(HID),
            full((HID, HID)), row(HID),
            full((HID, EMBED * NTOK)), row(EMBED * NTOK),
            full((EMBED * NTOK, HID)), row(HID),
            full((HID, HID)), row(HID),
            full((HID, IN_DIM)), row(IN_DIM),
            full((KCODES, EMBED)),
            full((EMBED, KCODES)),
        ],
        out_specs=[
            pl.BlockSpec((BLK, IN_DIM), lambda i: (i, 0)),
            pl.BlockSpec((BLK, NTOK), lambda i: (i, 0)),
            pl.BlockSpec((1, 1, 128), lambda i: (i, 0, 0)),
        ],
        out_shape=[
            jax.ShapeDtypeStruct((BATCH, IN_DIM), jnp.float32),
            jax.ShapeDtypeStruct((BATCH, NTOK), jnp.int32),
            jax.ShapeDtypeStruct((GRID, 1, 128), jnp.float32),
        ],
        scratch_shapes=[pltpu.VMEM((BLK, EMBED * NTOK), jnp.float32)],
        compiler_params=pltpu.CompilerParams(
            dimension_semantics=("parallel",),
        ),
    )(x,
      enc_W1, enc_b1.reshape(1, HID),
      enc_W2, enc_b2.reshape(1, HID),
      enc_W3, enc_b3.reshape(1, EMBED * NTOK),
      dec_W1, dec_b1.reshape(1, HID),
      dec_W2, dec_b2.reshape(1, HID),
      dec_W3, dec_b3.reshape(1, IN_DIM),
      codebook, cbT)

    commit_loss = jnp.sum(closs[:, 0, 0]) / jnp.float32(BATCH * NTOK * EMBED)
    return rec, idx, commit_loss
